# baseline (speedup 1.0000x reference)
import functools

import jax
import jax.numpy as jnp
from jax import lax
from jax.experimental import pallas as pl
from jax.experimental.pallas import tpu as pltpu
from jax.experimental.pallas import tpu_sc as plsc

N = 10000
E = 320000
D = 128

NC = 2
NS = 16
NW = NC * NS

CHUNK = 72
NBUF = 2
C0 = 140
C1 = 140
EC0 = NS * C0 * CHUNK
EC1 = NS * C1 * CHUNK
E_PAD = EC0 + EC1
ROWS_PER_TILE = N // NS
N_ACC = N + 16

E_PER_TILE_DEG = E // NW


def _sc_mesh():
  return plsc.VectorSubcoreMesh(core_axis_name="c", subcore_axis_name="s")


def _deg_kernel_body(dst_hbm, out_hbm, dst_v, deg_v):
  cid = lax.axis_index("c")
  sid = lax.axis_index("s")
  wid = cid * NS + sid
  pltpu.sync_copy(dst_hbm.at[wid], dst_v)

  zeros = jnp.zeros((16,), jnp.float32)

  def zbody(i, _):
    deg_v[pl.ds(i * 16, 16)] = zeros
    return ()

  lax.fori_loop(0, N // 16, zbody, ())

  ones = jnp.ones((16,), jnp.float32)

  def body(i, _):
    idx = dst_v[pl.ds(i * 16, 16)]
    plsc.addupdate_scatter(deg_v, [idx], ones)
    return ()

  lax.fori_loop(0, E_PER_TILE_DEG // 16, body, ())
  for g in range(GRID):
    pltpu.sync_copy(deg_v.at[pl.ds(g * BN, BN)], out_hbm.at[g, wid])


def _make_deg_kernel():
  return pl.kernel(
      _deg_kernel_body,
      out_type=jax.ShapeDtypeStruct((GRID, NW, BN), jnp.float32),
      mesh=_sc_mesh(),
      scratch_types=[
          pltpu.VMEM((E_PER_TILE_DEG,), jnp.int32),
          pltpu.VMEM((N,), jnp.float32),
      ],
      compiler_params=pltpu.CompilerParams(
          needs_layout_passes=False, use_tc_tiling_on_sc=False),
  )


def _agg_kernel_body(h_hbm, src_hbm, dst_hbm, out_hbm,
                     src_v, dst_v, rows_v, acc_sh, *sems):
  gsems = sems[:NBUF]
  ssems = sems[NBUF:]
  cid = lax.axis_index("c")
  sid = lax.axis_index("s")
  wid = cid * NS + sid
  r0 = sid * ROWS_PER_TILE
  nC = jnp.where(cid == 0, C0, C1)
  pltpu.sync_copy(h_hbm.at[pl.ds(r0, ROWS_PER_TILE)],
                  acc_sh.at[pl.ds(r0, ROWS_PER_TILE)])
  pltpu.sync_copy(src_hbm.at[wid, pl.ds(0, C1)], src_v.at[pl.ds(0, C1)])
  pltpu.sync_copy(dst_hbm.at[wid, pl.ds(0, C1)], dst_v.at[pl.ds(0, C1)])

  if C0 > C1:
    @pl.when(cid == 0)
    def _():
      pltpu.sync_copy(src_hbm.at[wid, pl.ds(C1, C0 - C1)],
                      src_v.at[pl.ds(C1, C0 - C1)])
      pltpu.sync_copy(dst_hbm.at[wid, pl.ds(C1, C0 - C1)],
                      dst_v.at[pl.ds(C1, C0 - C1)])

  plsc.subcore_barrier()

  def g_start(c, b):
    pltpu.async_copy(h_hbm.at[src_v.at[c]], rows_v.at[b], gsems[b])

  def g_wait(c, b):
    pltpu.make_async_copy(h_hbm.at[src_v.at[c]], rows_v.at[b],
                          gsems[b]).wait()

  def s_start(c, b):
    pltpu.async_copy(rows_v.at[b], acc_sh.at[dst_v.at[c]], ssems[b],
                     add=True)

  def s_wait(c, b):
    pltpu.make_async_copy(rows_v.at[b], acc_sh.at[dst_v.at[c]],
                          ssems[b]).wait()

  g_start(0, 0)

  def group(g, _):
    for b in range(NBUF):
      c = g * NBUF + b
      g_wait(c, b)
      nb = (b + 1) % NBUF
      nc = c + 1

      @pl.when(nc < nC)
      def _():
        @pl.when(c >= 1)
        def _():
          s_wait(c - 1, nb)
        g_start(nc, nb)

      s_start(c, b)
    return ()

  lax.fori_loop(0, jnp.where(cid == 0, C0 // NBUF, C1 // NBUF), group, ())
  s_wait(nC - NBUF, 0)
  s_wait(nC - 1, 1)
  plsc.subcore_barrier()
  pltpu.sync_copy(acc_sh.at[pl.ds(r0, ROWS_PER_TILE)],
                  out_hbm.at[cid, pl.ds(r0, ROWS_PER_TILE)])


def _make_agg_kernel():
  return pl.kernel(
      _agg_kernel_body,
      out_type=jax.ShapeDtypeStruct((NC, N, D), jnp.float32),
      mesh=_sc_mesh(),
      scratch_types=(
          [pltpu.VMEM((C0, CHUNK), jnp.int32),
           pltpu.VMEM((C0, CHUNK), jnp.int32),
           pltpu.VMEM((NBUF, CHUNK, D), jnp.float32),
           pltpu.VMEM_SHARED((N_ACC, D), jnp.float32)]
          + [pltpu.SemaphoreType.DMA] * (2 * NBUF)),
      compiler_params=pltpu.CompilerParams(use_tc_tiling_on_sc=False),
  )


BN = 1000
GRID = N // BN


def _tc1_body(deg_ref, x_ref, w_ref, h_ref, dis_ref):
  deg = jnp.sum(deg_ref[0], axis=0) + 1.0
  dis = lax.rsqrt(deg)
  h = jnp.dot(x_ref[...], w_ref[...], preferred_element_type=jnp.float32)
  h_ref[...] = h * dis[:, None]
  dis_ref[...] = dis[:, None]


def _tc1(deg_parts, x, w1):
  return pl.pallas_call(
      _tc1_body,
      grid=(GRID,),
      in_specs=[
          pl.BlockSpec((1, NW, BN), lambda i: (i, 0, 0)),
          pl.BlockSpec((BN, D), lambda i: (i, 0)),
          pl.BlockSpec((D, D), lambda i: (0, 0)),
      ],
      out_specs=[
          pl.BlockSpec((BN, D), lambda i: (i, 0)),
          pl.BlockSpec((BN, 1), lambda i: (i, 0)),
      ],
      out_shape=[
          jax.ShapeDtypeStruct((N, D), jnp.float32),
          jax.ShapeDtypeStruct((N, 1), jnp.float32),
      ],
  )(deg_parts, x, w1)


def _tc2_body(a_ref, h1_ref, dis_ref, w_ref, b_ref, out_ref):
  dis = dis_ref[...]
  agg = a_ref[0] + a_ref[1] - h1_ref[...]
  o1 = jnp.maximum(agg * dis + b_ref[...], 0.0)
  out_ref[...] = jnp.dot(o1, w_ref[...],
                         preferred_element_type=jnp.float32) * dis


def _tc2(agg_parts, h1p, dis, w2, b1):
  return pl.pallas_call(
      _tc2_body,
      grid=(GRID,),
      in_specs=[
          pl.BlockSpec((NC, BN, D), lambda i: (0, i, 0)),
          pl.BlockSpec((BN, D), lambda i: (i, 0)),
          pl.BlockSpec((BN, 1), lambda i: (i, 0)),
          pl.BlockSpec((D, D), lambda i: (0, 0)),
          pl.BlockSpec((1, D), lambda i: (0, 0)),
      ],
      out_specs=pl.BlockSpec((BN, D), lambda i: (i, 0)),
      out_shape=jax.ShapeDtypeStruct((N, D), jnp.float32),
  )(agg_parts, h1p, dis, w2, b1)


def _tc3_body(b_ref, h2_ref, dis_ref, bias_ref, out_ref):
  agg = b_ref[0] + b_ref[1] - h2_ref[...]
  out_ref[...] = agg * dis_ref[...] + bias_ref[...]


def _tc3(agg_parts, h2p, dis, b2):
  return pl.pallas_call(
      _tc3_body,
      grid=(GRID,),
      in_specs=[
          pl.BlockSpec((NC, BN, D), lambda i: (0, i, 0)),
          pl.BlockSpec((BN, D), lambda i: (i, 0)),
          pl.BlockSpec((BN, 1), lambda i: (i, 0)),
          pl.BlockSpec((1, D), lambda i: (0, 0)),
      ],
      out_specs=pl.BlockSpec((BN, D), lambda i: (i, 0)),
      out_shape=jax.ShapeDtypeStruct((N, D), jnp.float32),
  )(agg_parts, h2p, dis, b2)


@jax.jit
def kernel(x, edge_index, W1, b1, W2, b2):
  src = edge_index[0]
  dst = edge_index[1]
  pad = E_PAD - E
  src_p = jnp.concatenate([src, jnp.zeros((pad,), jnp.int32)])
  dst_p = jnp.concatenate([dst, jnp.full((pad,), N, jnp.int32)])

  def _split(a, fill):
    a0 = a[:EC0].reshape(NS, C0, CHUNK)
    a1 = jnp.pad(a[EC0:].reshape(NS, C1, CHUNK),
                 ((0, 0), (0, C0 - C1), (0, 0)), constant_values=fill)
    return jnp.concatenate([a0, a1], axis=0)

  src_p = _split(src_p, 0)
  dst_p = _split(dst_p, N)
  dst_deg = dst.reshape(NW, E_PER_TILE_DEG)

  deg_parts = _make_deg_kernel()(dst_deg)
  h1p, dis = _tc1(deg_parts, x, W1)
  agg1 = _make_agg_kernel()(h1p, src_p, dst_p)
  h2p = _tc2(agg1, h1p, dis, W2, b1.reshape(1, D))
  agg2 = _make_agg_kernel()(h2p, src_p, dst_p)
  return _tc3(agg2, h2p, dis, b2.reshape(1, D))


# speedup vs baseline: 1.3359x; 1.3359x over previous
import functools

import jax
import jax.numpy as jnp
from jax import lax
from jax.experimental import pallas as pl
from jax.experimental.pallas import tpu as pltpu
from jax.experimental.pallas import tpu_sc as plsc

N = 10000
E = 320000
D = 128

NC = 2
NS = 16
NW = NC * NS

CHUNK = 72
NBUF = 2
C0 = 160
C1 = 118
EC0 = NS * C0 * CHUNK
EC1 = NS * C1 * CHUNK
E_PAD = EC0 + EC1
ROWS_PER_TILE = N // NS
N_ACC = N + 16

E_PER_TILE_DEG = E // NW


def _sc_mesh():
  return plsc.VectorSubcoreMesh(core_axis_name="c", subcore_axis_name="s")


def _deg_kernel_body(dst_hbm, out_hbm, dst_v, deg_v):
  cid = lax.axis_index("c")
  sid = lax.axis_index("s")
  wid = cid * NS + sid
  pltpu.sync_copy(dst_hbm.at[wid], dst_v)

  zeros = jnp.zeros((16,), jnp.float32)

  def zbody(i, _):
    deg_v[pl.ds(i * 16, 16)] = zeros
    return ()

  lax.fori_loop(0, N // 16, zbody, ())

  ones = jnp.ones((16,), jnp.float32)

  def body(i, _):
    idx = dst_v[pl.ds(i * 16, 16)]
    plsc.addupdate_scatter(deg_v, [idx], ones)
    return ()

  lax.fori_loop(0, E_PER_TILE_DEG // 16, body, ())
  for g in range(GRID):
    pltpu.sync_copy(deg_v.at[pl.ds(g * BN, BN)], out_hbm.at[g, wid])


def _make_deg_kernel():
  return pl.kernel(
      _deg_kernel_body,
      out_type=jax.ShapeDtypeStruct((GRID, NW, BN), jnp.float32),
      mesh=_sc_mesh(),
      scratch_types=[
          pltpu.VMEM((E_PER_TILE_DEG,), jnp.int32),
          pltpu.VMEM((N,), jnp.float32),
      ],
      compiler_params=pltpu.CompilerParams(
          needs_layout_passes=False, use_tc_tiling_on_sc=False),
  )


def _agg_kernel_body(h_hbm, src_hbm, dst_hbm, out_hbm,
                     src_v, dst_v, rows_v, acc_sh, *sems):
  gsems = sems[:NBUF]
  ssems = sems[NBUF:]
  cid = lax.axis_index("c")
  sid = lax.axis_index("s")
  wid = cid * NS + sid
  r0 = sid * ROWS_PER_TILE
  nC = jnp.where(cid == 0, C0, C1)
  pltpu.sync_copy(h_hbm.at[pl.ds(r0, ROWS_PER_TILE)],
                  acc_sh.at[pl.ds(r0, ROWS_PER_TILE)])
  pltpu.sync_copy(src_hbm.at[wid, pl.ds(0, C1)], src_v.at[pl.ds(0, C1)])
  pltpu.sync_copy(dst_hbm.at[wid, pl.ds(0, C1)], dst_v.at[pl.ds(0, C1)])

  if C0 > C1:
    @pl.when(cid == 0)
    def _():
      pltpu.sync_copy(src_hbm.at[wid, pl.ds(C1, C0 - C1)],
                      src_v.at[pl.ds(C1, C0 - C1)])
      pltpu.sync_copy(dst_hbm.at[wid, pl.ds(C1, C0 - C1)],
                      dst_v.at[pl.ds(C1, C0 - C1)])

  plsc.subcore_barrier()

  def g_start(c, b):
    pltpu.async_copy(h_hbm.at[src_v.at[c]], rows_v.at[b], gsems[b])

  def g_wait(c, b):
    pltpu.make_async_copy(h_hbm.at[src_v.at[c]], rows_v.at[b],
                          gsems[b]).wait()

  def s_start(c, b):
    pltpu.async_copy(rows_v.at[b], acc_sh.at[dst_v.at[c]], ssems[b],
                     add=True)

  def s_wait(c, b):
    pltpu.make_async_copy(rows_v.at[b], acc_sh.at[dst_v.at[c]],
                          ssems[b]).wait()

  g_start(0, 0)

  def group(g, _):
    for b in range(NBUF):
      c = g * NBUF + b
      g_wait(c, b)
      nb = (b + 1) % NBUF
      nc = c + 1

      @pl.when(nc < nC)
      def _():
        @pl.when(c >= 1)
        def _():
          s_wait(c - 1, nb)
        g_start(nc, nb)

      s_start(c, b)
    return ()

  lax.fori_loop(0, jnp.where(cid == 0, C0 // NBUF, C1 // NBUF), group, ())
  s_wait(nC - NBUF, 0)
  s_wait(nC - 1, 1)
  plsc.subcore_barrier()
  pltpu.sync_copy(acc_sh.at[pl.ds(r0, ROWS_PER_TILE)],
                  out_hbm.at[cid, pl.ds(r0, ROWS_PER_TILE)])


def _make_agg_kernel():
  return pl.kernel(
      _agg_kernel_body,
      out_type=jax.ShapeDtypeStruct((NC, N, D), jnp.float32),
      mesh=_sc_mesh(),
      scratch_types=(
          [pltpu.VMEM((C0, CHUNK), jnp.int32),
           pltpu.VMEM((C0, CHUNK), jnp.int32),
           pltpu.VMEM((NBUF, CHUNK, D), jnp.float32),
           pltpu.VMEM_SHARED((N_ACC, D), jnp.float32)]
          + [pltpu.SemaphoreType.DMA] * (2 * NBUF)),
      compiler_params=pltpu.CompilerParams(use_tc_tiling_on_sc=False),
  )


BN = 1000
GRID = N // BN


def _tc1_body(deg_ref, x_ref, w_ref, h_ref, dis_ref):
  deg = jnp.sum(deg_ref[0], axis=0) + 1.0
  dis = lax.rsqrt(deg)
  h = jnp.dot(x_ref[...], w_ref[...], preferred_element_type=jnp.float32)
  h_ref[...] = h * dis[:, None]
  dis_ref[...] = dis[:, None]


def _tc1(deg_parts, x, w1):
  return pl.pallas_call(
      _tc1_body,
      grid=(GRID,),
      in_specs=[
          pl.BlockSpec((1, NW, BN), lambda i: (i, 0, 0)),
          pl.BlockSpec((BN, D), lambda i: (i, 0)),
          pl.BlockSpec((D, D), lambda i: (0, 0)),
      ],
      out_specs=[
          pl.BlockSpec((BN, D), lambda i: (i, 0)),
          pl.BlockSpec((BN, 1), lambda i: (i, 0)),
      ],
      out_shape=[
          jax.ShapeDtypeStruct((N, D), jnp.float32),
          jax.ShapeDtypeStruct((N, 1), jnp.float32),
      ],
  )(deg_parts, x, w1)


def _tc2_body(a_ref, h1_ref, dis_ref, w_ref, b_ref, out_ref):
  dis = dis_ref[...]
  agg = a_ref[0] + a_ref[1] - h1_ref[...]
  o1 = jnp.maximum(agg * dis + b_ref[...], 0.0)
  out_ref[...] = jnp.dot(o1, w_ref[...],
                         preferred_element_type=jnp.float32) * dis


def _tc2(agg_parts, h1p, dis, w2, b1):
  return pl.pallas_call(
      _tc2_body,
      grid=(GRID,),
      in_specs=[
          pl.BlockSpec((NC, BN, D), lambda i: (0, i, 0)),
          pl.BlockSpec((BN, D), lambda i: (i, 0)),
          pl.BlockSpec((BN, 1), lambda i: (i, 0)),
          pl.BlockSpec((D, D), lambda i: (0, 0)),
          pl.BlockSpec((1, D), lambda i: (0, 0)),
      ],
      out_specs=pl.BlockSpec((BN, D), lambda i: (i, 0)),
      out_shape=jax.ShapeDtypeStruct((N, D), jnp.float32),
  )(agg_parts, h1p, dis, w2, b1)


def _tc3_body(b_ref, h2_ref, dis_ref, bias_ref, out_ref):
  agg = b_ref[0] + b_ref[1] - h2_ref[...]
  out_ref[...] = agg * dis_ref[...] + bias_ref[...]


def _tc3(agg_parts, h2p, dis, b2):
  return pl.pallas_call(
      _tc3_body,
      grid=(GRID,),
      in_specs=[
          pl.BlockSpec((NC, BN, D), lambda i: (0, i, 0)),
          pl.BlockSpec((BN, D), lambda i: (i, 0)),
          pl.BlockSpec((BN, 1), lambda i: (i, 0)),
          pl.BlockSpec((1, D), lambda i: (0, 0)),
      ],
      out_specs=pl.BlockSpec((BN, D), lambda i: (i, 0)),
      out_shape=jax.ShapeDtypeStruct((N, D), jnp.float32),
  )(agg_parts, h2p, dis, b2)


@jax.jit
def kernel(x, edge_index, W1, b1, W2, b2):
  src = edge_index[0]
  dst = edge_index[1]
  pad = E_PAD - E
  src_p = jnp.concatenate([src, jnp.zeros((pad,), jnp.int32)])
  dst_p = jnp.concatenate([dst, jnp.full((pad,), N, jnp.int32)])

  def _split(a, fill):
    a0 = a[:EC0].reshape(NS, C0, CHUNK)
    a1 = jnp.pad(a[EC0:].reshape(NS, C1, CHUNK),
                 ((0, 0), (0, C0 - C1), (0, 0)), constant_values=fill)
    return jnp.concatenate([a0, a1], axis=0)

  src_p = _split(src_p, 0)
  dst_p = _split(dst_p, N)
  dst_deg = dst.reshape(NW, E_PER_TILE_DEG)

  deg_parts = _make_deg_kernel()(dst_deg)
  h1p, dis = _tc1(deg_parts, x, W1)
  agg1 = _make_agg_kernel()(h1p, src_p, dst_p)
  h2p = _tc2(agg1, h1p, dis, W2, b1.reshape(1, D))
  agg2 = _make_agg_kernel()(h2p, src_p, dst_p)
  return _tc3(agg2, h2p, dis, b2.reshape(1, D))


# speedup vs baseline: 1.4125x; 1.0573x over previous
import functools

import jax
import jax.numpy as jnp
from jax import lax
from jax.experimental import pallas as pl
from jax.experimental.pallas import tpu as pltpu
from jax.experimental.pallas import tpu_sc as plsc

N = 10000
E = 320000
D = 128

NC = 2
NS = 16
NW = NC * NS

CHUNK = 72
NBUF = 2
C0 = 152
C1 = 126
EC0 = NS * C0 * CHUNK
EC1 = NS * C1 * CHUNK
E_PAD = EC0 + EC1
ROWS_PER_TILE = N // NS
N_ACC = N + 16

E_PER_TILE_DEG = E // NW


def _sc_mesh():
  return plsc.VectorSubcoreMesh(core_axis_name="c", subcore_axis_name="s")


def _deg_kernel_body(dst_hbm, out_hbm, dst_v, deg_v):
  cid = lax.axis_index("c")
  sid = lax.axis_index("s")
  wid = cid * NS + sid
  pltpu.sync_copy(dst_hbm.at[wid], dst_v)

  zeros = jnp.zeros((16,), jnp.float32)

  def zbody(i, _):
    deg_v[pl.ds(i * 16, 16)] = zeros
    return ()

  lax.fori_loop(0, N // 16, zbody, ())

  ones = jnp.ones((16,), jnp.float32)

  def body(i, _):
    idx = dst_v[pl.ds(i * 16, 16)]
    plsc.addupdate_scatter(deg_v, [idx], ones)
    return ()

  lax.fori_loop(0, E_PER_TILE_DEG // 16, body, ())
  for g in range(GRID):
    pltpu.sync_copy(deg_v.at[pl.ds(g * BN, BN)], out_hbm.at[g, wid])


def _make_deg_kernel():
  return pl.kernel(
      _deg_kernel_body,
      out_type=jax.ShapeDtypeStruct((GRID, NW, BN), jnp.float32),
      mesh=_sc_mesh(),
      scratch_types=[
          pltpu.VMEM((E_PER_TILE_DEG,), jnp.int32),
          pltpu.VMEM((N,), jnp.float32),
      ],
      compiler_params=pltpu.CompilerParams(
          needs_layout_passes=False, use_tc_tiling_on_sc=False),
  )


def _agg_kernel_body(h_hbm, src_hbm, dst_hbm, out_hbm,
                     src_v, dst_v, rows_v, acc_sh, *sems):
  gsems = sems[:NBUF]
  ssems = sems[NBUF:]
  cid = lax.axis_index("c")
  sid = lax.axis_index("s")
  wid = cid * NS + sid
  r0 = sid * ROWS_PER_TILE
  nC = jnp.where(cid == 0, C0, C1)
  pltpu.sync_copy(h_hbm.at[pl.ds(r0, ROWS_PER_TILE)],
                  acc_sh.at[pl.ds(r0, ROWS_PER_TILE)])
  pltpu.sync_copy(src_hbm.at[wid, pl.ds(0, C1)], src_v.at[pl.ds(0, C1)])
  pltpu.sync_copy(dst_hbm.at[wid, pl.ds(0, C1)], dst_v.at[pl.ds(0, C1)])

  if C0 > C1:
    @pl.when(cid == 0)
    def _():
      pltpu.sync_copy(src_hbm.at[wid, pl.ds(C1, C0 - C1)],
                      src_v.at[pl.ds(C1, C0 - C1)])
      pltpu.sync_copy(dst_hbm.at[wid, pl.ds(C1, C0 - C1)],
                      dst_v.at[pl.ds(C1, C0 - C1)])

  plsc.subcore_barrier()

  def g_start(c, b):
    pltpu.async_copy(h_hbm.at[src_v.at[c]], rows_v.at[b], gsems[b])

  def g_wait(c, b):
    pltpu.make_async_copy(h_hbm.at[src_v.at[c]], rows_v.at[b],
                          gsems[b]).wait()

  def s_start(c, b):
    pltpu.async_copy(rows_v.at[b], acc_sh.at[dst_v.at[c]], ssems[b],
                     add=True)

  def s_wait(c, b):
    pltpu.make_async_copy(rows_v.at[b], acc_sh.at[dst_v.at[c]],
                          ssems[b]).wait()

  g_start(0, 0)

  def group(g, _):
    for b in range(NBUF):
      c = g * NBUF + b
      g_wait(c, b)
      nb = (b + 1) % NBUF
      nc = c + 1

      @pl.when(nc < nC)
      def _():
        @pl.when(c >= 1)
        def _():
          s_wait(c - 1, nb)
        g_start(nc, nb)

      s_start(c, b)
    return ()

  lax.fori_loop(0, jnp.where(cid == 0, C0 // NBUF, C1 // NBUF), group, ())
  s_wait(nC - NBUF, 0)
  s_wait(nC - 1, 1)
  plsc.subcore_barrier()
  pltpu.sync_copy(acc_sh.at[pl.ds(r0, ROWS_PER_TILE)],
                  out_hbm.at[cid, pl.ds(r0, ROWS_PER_TILE)])


def _make_agg_kernel():
  return pl.kernel(
      _agg_kernel_body,
      out_type=jax.ShapeDtypeStruct((NC, N, D), jnp.float32),
      mesh=_sc_mesh(),
      scratch_types=(
          [pltpu.VMEM((C0, CHUNK), jnp.int32),
           pltpu.VMEM((C0, CHUNK), jnp.int32),
           pltpu.VMEM((NBUF, CHUNK, D), jnp.float32),
           pltpu.VMEM_SHARED((N_ACC, D), jnp.float32)]
          + [pltpu.SemaphoreType.DMA] * (2 * NBUF)),
      compiler_params=pltpu.CompilerParams(use_tc_tiling_on_sc=False),
  )


BN = 1000
GRID = N // BN


def _tc1_body(deg_ref, x_ref, w_ref, h_ref, dis_ref):
  deg = jnp.sum(deg_ref[0], axis=0) + 1.0
  dis = lax.rsqrt(deg)
  h = jnp.dot(x_ref[...], w_ref[...], preferred_element_type=jnp.float32)
  h_ref[...] = h * dis[:, None]
  dis_ref[...] = dis[:, None]


def _tc1(deg_parts, x, w1):
  return pl.pallas_call(
      _tc1_body,
      grid=(GRID,),
      in_specs=[
          pl.BlockSpec((1, NW, BN), lambda i: (i, 0, 0)),
          pl.BlockSpec((BN, D), lambda i: (i, 0)),
          pl.BlockSpec((D, D), lambda i: (0, 0)),
      ],
      out_specs=[
          pl.BlockSpec((BN, D), lambda i: (i, 0)),
          pl.BlockSpec((BN, 1), lambda i: (i, 0)),
      ],
      out_shape=[
          jax.ShapeDtypeStruct((N, D), jnp.float32),
          jax.ShapeDtypeStruct((N, 1), jnp.float32),
      ],
  )(deg_parts, x, w1)


def _tc2_body(a_ref, h1_ref, dis_ref, w_ref, b_ref, out_ref):
  dis = dis_ref[...]
  agg = a_ref[0] + a_ref[1] - h1_ref[...]
  o1 = jnp.maximum(agg * dis + b_ref[...], 0.0)
  out_ref[...] = jnp.dot(o1, w_ref[...],
                         preferred_element_type=jnp.float32) * dis


def _tc2(agg_parts, h1p, dis, w2, b1):
  return pl.pallas_call(
      _tc2_body,
      grid=(GRID,),
      in_specs=[
          pl.BlockSpec((NC, BN, D), lambda i: (0, i, 0)),
          pl.BlockSpec((BN, D), lambda i: (i, 0)),
          pl.BlockSpec((BN, 1), lambda i: (i, 0)),
          pl.BlockSpec((D, D), lambda i: (0, 0)),
          pl.BlockSpec((1, D), lambda i: (0, 0)),
      ],
      out_specs=pl.BlockSpec((BN, D), lambda i: (i, 0)),
      out_shape=jax.ShapeDtypeStruct((N, D), jnp.float32),
  )(agg_parts, h1p, dis, w2, b1)


def _tc3_body(b_ref, h2_ref, dis_ref, bias_ref, out_ref):
  agg = b_ref[0] + b_ref[1] - h2_ref[...]
  out_ref[...] = agg * dis_ref[...] + bias_ref[...]


def _tc3(agg_parts, h2p, dis, b2):
  return pl.pallas_call(
      _tc3_body,
      grid=(GRID,),
      in_specs=[
          pl.BlockSpec((NC, BN, D), lambda i: (0, i, 0)),
          pl.BlockSpec((BN, D), lambda i: (i, 0)),
          pl.BlockSpec((BN, 1), lambda i: (i, 0)),
          pl.BlockSpec((1, D), lambda i: (0, 0)),
      ],
      out_specs=pl.BlockSpec((BN, D), lambda i: (i, 0)),
      out_shape=jax.ShapeDtypeStruct((N, D), jnp.float32),
  )(agg_parts, h2p, dis, b2)


@jax.jit
def kernel(x, edge_index, W1, b1, W2, b2):
  src = edge_index[0]
  dst = edge_index[1]
  pad = E_PAD - E
  src_p = jnp.concatenate([src, jnp.zeros((pad,), jnp.int32)])
  dst_p = jnp.concatenate([dst, jnp.full((pad,), N, jnp.int32)])

  def _split(a, fill):
    a0 = a[:EC0].reshape(NS, C0, CHUNK)
    a1 = jnp.pad(a[EC0:].reshape(NS, C1, CHUNK),
                 ((0, 0), (0, C0 - C1), (0, 0)), constant_values=fill)
    return jnp.concatenate([a0, a1], axis=0)

  src_p = _split(src_p, 0)
  dst_p = _split(dst_p, N)
  dst_deg = dst.reshape(NW, E_PER_TILE_DEG)

  deg_parts = _make_deg_kernel()(dst_deg)
  h1p, dis = _tc1(deg_parts, x, W1)
  agg1 = _make_agg_kernel()(h1p, src_p, dst_p)
  h2p = _tc2(agg1, h1p, dis, W2, b1.reshape(1, D))
  agg2 = _make_agg_kernel()(h2p, src_p, dst_p)
  return _tc3(agg2, h2p, dis, b2.reshape(1, D))


# speedup vs baseline: 1.4461x; 1.0238x over previous
import functools

import jax
import jax.numpy as jnp
from jax import lax
from jax.experimental import pallas as pl
from jax.experimental.pallas import tpu as pltpu
from jax.experimental.pallas import tpu_sc as plsc

N = 10000
E = 320000
D = 128

NC = 2
NS = 16
NW = NC * NS

CHUNK = 72
NBUF = 2
C0 = 146
C1 = 132
EC0 = NS * C0 * CHUNK
EC1 = NS * C1 * CHUNK
E_PAD = EC0 + EC1
ROWS_PER_TILE = N // NS
N_ACC = N + 16

E_PER_TILE_DEG = E // NW


def _sc_mesh():
  return plsc.VectorSubcoreMesh(core_axis_name="c", subcore_axis_name="s")


def _deg_kernel_body(dst_hbm, out_hbm, dst_v, deg_v):
  cid = lax.axis_index("c")
  sid = lax.axis_index("s")
  wid = cid * NS + sid
  pltpu.sync_copy(dst_hbm.at[wid], dst_v)

  zeros = jnp.zeros((16,), jnp.float32)

  def zbody(i, _):
    deg_v[pl.ds(i * 16, 16)] = zeros
    return ()

  lax.fori_loop(0, N // 16, zbody, ())

  ones = jnp.ones((16,), jnp.float32)

  def body(i, _):
    idx = dst_v[pl.ds(i * 16, 16)]
    plsc.addupdate_scatter(deg_v, [idx], ones)
    return ()

  lax.fori_loop(0, E_PER_TILE_DEG // 16, body, ())
  for g in range(GRID):
    pltpu.sync_copy(deg_v.at[pl.ds(g * BN, BN)], out_hbm.at[g, wid])


def _make_deg_kernel():
  return pl.kernel(
      _deg_kernel_body,
      out_type=jax.ShapeDtypeStruct((GRID, NW, BN), jnp.float32),
      mesh=_sc_mesh(),
      scratch_types=[
          pltpu.VMEM((E_PER_TILE_DEG,), jnp.int32),
          pltpu.VMEM((N,), jnp.float32),
      ],
      compiler_params=pltpu.CompilerParams(
          needs_layout_passes=False, use_tc_tiling_on_sc=False),
  )


def _agg_kernel_body(h_hbm, src_hbm, dst_hbm, out_hbm,
                     src_v, dst_v, rows_v, acc_sh, *sems):
  gsems = sems[:NBUF]
  ssems = sems[NBUF:]
  cid = lax.axis_index("c")
  sid = lax.axis_index("s")
  wid = cid * NS + sid
  r0 = sid * ROWS_PER_TILE
  nC = jnp.where(cid == 0, C0, C1)
  pltpu.sync_copy(h_hbm.at[pl.ds(r0, ROWS_PER_TILE)],
                  acc_sh.at[pl.ds(r0, ROWS_PER_TILE)])
  pltpu.sync_copy(src_hbm.at[wid, pl.ds(0, C1)], src_v.at[pl.ds(0, C1)])
  pltpu.sync_copy(dst_hbm.at[wid, pl.ds(0, C1)], dst_v.at[pl.ds(0, C1)])

  if C0 > C1:
    @pl.when(cid == 0)
    def _():
      pltpu.sync_copy(src_hbm.at[wid, pl.ds(C1, C0 - C1)],
                      src_v.at[pl.ds(C1, C0 - C1)])
      pltpu.sync_copy(dst_hbm.at[wid, pl.ds(C1, C0 - C1)],
                      dst_v.at[pl.ds(C1, C0 - C1)])

  plsc.subcore_barrier()

  def g_start(c, b):
    pltpu.async_copy(h_hbm.at[src_v.at[c]], rows_v.at[b], gsems[b])

  def g_wait(c, b):
    pltpu.make_async_copy(h_hbm.at[src_v.at[c]], rows_v.at[b],
                          gsems[b]).wait()

  def s_start(c, b):
    pltpu.async_copy(rows_v.at[b], acc_sh.at[dst_v.at[c]], ssems[b],
                     add=True)

  def s_wait(c, b):
    pltpu.make_async_copy(rows_v.at[b], acc_sh.at[dst_v.at[c]],
                          ssems[b]).wait()

  g_start(0, 0)

  def group(g, _):
    for b in range(NBUF):
      c = g * NBUF + b
      g_wait(c, b)
      nb = (b + 1) % NBUF
      nc = c + 1

      @pl.when(nc < nC)
      def _():
        @pl.when(c >= 1)
        def _():
          s_wait(c - 1, nb)
        g_start(nc, nb)

      s_start(c, b)
    return ()

  lax.fori_loop(0, jnp.where(cid == 0, C0 // NBUF, C1 // NBUF), group, ())
  s_wait(nC - NBUF, 0)
  s_wait(nC - 1, 1)
  plsc.subcore_barrier()
  pltpu.sync_copy(acc_sh.at[pl.ds(r0, ROWS_PER_TILE)],
                  out_hbm.at[cid, pl.ds(r0, ROWS_PER_TILE)])


def _make_agg_kernel():
  return pl.kernel(
      _agg_kernel_body,
      out_type=jax.ShapeDtypeStruct((NC, N, D), jnp.float32),
      mesh=_sc_mesh(),
      scratch_types=(
          [pltpu.VMEM((C0, CHUNK), jnp.int32),
           pltpu.VMEM((C0, CHUNK), jnp.int32),
           pltpu.VMEM((NBUF, CHUNK, D), jnp.float32),
           pltpu.VMEM_SHARED((N_ACC, D), jnp.float32)]
          + [pltpu.SemaphoreType.DMA] * (2 * NBUF)),
      compiler_params=pltpu.CompilerParams(use_tc_tiling_on_sc=False),
  )


BN = 1000
GRID = N // BN


def _tc1_body(deg_ref, x_ref, w_ref, h_ref, dis_ref):
  deg = jnp.sum(deg_ref[0], axis=0) + 1.0
  dis = lax.rsqrt(deg)
  h = jnp.dot(x_ref[...], w_ref[...], preferred_element_type=jnp.float32)
  h_ref[...] = h * dis[:, None]
  dis_ref[...] = dis[:, None]


def _tc1(deg_parts, x, w1):
  return pl.pallas_call(
      _tc1_body,
      grid=(GRID,),
      in_specs=[
          pl.BlockSpec((1, NW, BN), lambda i: (i, 0, 0)),
          pl.BlockSpec((BN, D), lambda i: (i, 0)),
          pl.BlockSpec((D, D), lambda i: (0, 0)),
      ],
      out_specs=[
          pl.BlockSpec((BN, D), lambda i: (i, 0)),
          pl.BlockSpec((BN, 1), lambda i: (i, 0)),
      ],
      out_shape=[
          jax.ShapeDtypeStruct((N, D), jnp.float32),
          jax.ShapeDtypeStruct((N, 1), jnp.float32),
      ],
  )(deg_parts, x, w1)


def _tc2_body(a_ref, h1_ref, dis_ref, w_ref, b_ref, out_ref):
  dis = dis_ref[...]
  agg = a_ref[0] + a_ref[1] - h1_ref[...]
  o1 = jnp.maximum(agg * dis + b_ref[...], 0.0)
  out_ref[...] = jnp.dot(o1, w_ref[...],
                         preferred_element_type=jnp.float32) * dis


def _tc2(agg_parts, h1p, dis, w2, b1):
  return pl.pallas_call(
      _tc2_body,
      grid=(GRID,),
      in_specs=[
          pl.BlockSpec((NC, BN, D), lambda i: (0, i, 0)),
          pl.BlockSpec((BN, D), lambda i: (i, 0)),
          pl.BlockSpec((BN, 1), lambda i: (i, 0)),
          pl.BlockSpec((D, D), lambda i: (0, 0)),
          pl.BlockSpec((1, D), lambda i: (0, 0)),
      ],
      out_specs=pl.BlockSpec((BN, D), lambda i: (i, 0)),
      out_shape=jax.ShapeDtypeStruct((N, D), jnp.float32),
  )(agg_parts, h1p, dis, w2, b1)


def _tc3_body(b_ref, h2_ref, dis_ref, bias_ref, out_ref):
  agg = b_ref[0] + b_ref[1] - h2_ref[...]
  out_ref[...] = agg * dis_ref[...] + bias_ref[...]


def _tc3(agg_parts, h2p, dis, b2):
  return pl.pallas_call(
      _tc3_body,
      grid=(GRID,),
      in_specs=[
          pl.BlockSpec((NC, BN, D), lambda i: (0, i, 0)),
          pl.BlockSpec((BN, D), lambda i: (i, 0)),
          pl.BlockSpec((BN, 1), lambda i: (i, 0)),
          pl.BlockSpec((1, D), lambda i: (0, 0)),
      ],
      out_specs=pl.BlockSpec((BN, D), lambda i: (i, 0)),
      out_shape=jax.ShapeDtypeStruct((N, D), jnp.float32),
  )(agg_parts, h2p, dis, b2)


@jax.jit
def kernel(x, edge_index, W1, b1, W2, b2):
  src = edge_index[0]
  dst = edge_index[1]
  pad = E_PAD - E
  src_p = jnp.concatenate([src, jnp.zeros((pad,), jnp.int32)])
  dst_p = jnp.concatenate([dst, jnp.full((pad,), N, jnp.int32)])

  def _split(a, fill):
    a0 = a[:EC0].reshape(NS, C0, CHUNK)
    a1 = jnp.pad(a[EC0:].reshape(NS, C1, CHUNK),
                 ((0, 0), (0, C0 - C1), (0, 0)), constant_values=fill)
    return jnp.concatenate([a0, a1], axis=0)

  src_p = _split(src_p, 0)
  dst_p = _split(dst_p, N)
  dst_deg = dst.reshape(NW, E_PER_TILE_DEG)

  deg_parts = _make_deg_kernel()(dst_deg)
  h1p, dis = _tc1(deg_parts, x, W1)
  agg1 = _make_agg_kernel()(h1p, src_p, dst_p)
  h2p = _tc2(agg1, h1p, dis, W2, b1.reshape(1, D))
  agg2 = _make_agg_kernel()(h2p, src_p, dst_p)
  return _tc3(agg2, h2p, dis, b2.reshape(1, D))
